# baseline (speedup 1.0000x reference)
import functools

import jax
import jax.numpy as jnp
from jax.experimental import pallas as pl

_BLKC = 200
_BLKB = 2048


def _body(lab_ref, out_ref):
    j = pl.program_id(0)
    cb = pl.program_id(1)
    ib = pl.program_id(2)
    row = lab_ref[pl.ds(j, 1), pl.ds(ib * _BLKB, _BLKB)]
    cls = jax.lax.broadcasted_iota(jnp.int32, (_BLKC, 1), 0) + cb * _BLKC
    out_ref[0] = (row == cls).astype(jnp.float32)


def kernel(label, N):
    n_cls = 1000
    b, l = label.shape
    lab_t = label.T
    out = pl.pallas_call(
        _body,
        grid=(l, n_cls // _BLKC, b // _BLKB),
        in_specs=[pl.BlockSpec((l, b), lambda j, cb, ib: (0, 0))],
        out_specs=pl.BlockSpec(
            (1, _BLKC, _BLKB), lambda j, cb, ib: (j, cb, ib)
        ),
        out_shape=jax.ShapeDtypeStruct((l, n_cls, b), jnp.float32),
    )(lab_t)
    return out.transpose(1, 2, 0)


# speedup vs baseline: 1.3217x; 1.3217x over previous
import functools

import jax
import jax.numpy as jnp
from jax.experimental import pallas as pl

_BLKC = 200


def _body(lab_ref, out_ref):
    j = pl.program_id(0)
    cb = pl.program_id(1)
    row = lab_ref[pl.ds(j, 1), :]
    cls = jax.lax.broadcasted_iota(jnp.int32, (_BLKC, 1), 0) + cb * _BLKC
    out_ref[0] = (row == cls).astype(jnp.float32)


def kernel(label, N):
    n_cls = 1000
    b, l = label.shape
    lab_t = label.T
    out = pl.pallas_call(
        _body,
        grid=(l, n_cls // _BLKC),
        in_specs=[pl.BlockSpec((l, b), lambda j, cb: (0, 0))],
        out_specs=pl.BlockSpec((1, _BLKC, b), lambda j, cb: (j, cb, 0)),
        out_shape=jax.ShapeDtypeStruct((l, n_cls, b), jnp.float32),
    )(lab_t)
    return out.transpose(1, 2, 0)
